# baseline (speedup 1.0000x reference)
import functools

import jax
import jax.numpy as jnp
import numpy as np
from jax import lax
from jax.experimental import pallas as pl
from jax.experimental.pallas import tpu as pltpu
from jax.experimental.pallas import tpu_sc as plsc

N = 10000
E = 320000
D_IN = 128
HID = 64
C = 32
T = 3.0
K_TAYLOR = 5
E_NEG_T = float(np.exp(-T))

NC = 2
NS = 16
NW = NC * NS
U = 256
E2 = 327680
PAD = E2 - E
ROWS = E2 // U
UPT = ROWS // NW
GD = 8
ROUNDS_D = UPT // GD
NP = N + 16
RB = 624
ZB = 64

_PAD_SRC = np.arange(PAD, dtype=np.int32) % 16
_PAD_DST = N + (np.arange(PAD, dtype=np.int32) % 16)
_PAD_W = np.zeros((PAD,), dtype=np.float32)


def _mesh():
    return plsc.VectorSubcoreMesh(core_axis_name="c", subcore_axis_name="s")


_SC_PARAMS = pltpu.CompilerParams(use_tc_tiling_on_sc=False)


@functools.partial(
    pl.kernel,
    out_type=(
        jax.ShapeDtypeStruct((NC, N), jnp.float32),
        jax.ShapeDtypeStruct((NC, N), jnp.float32),
    ),
    mesh=_mesh(),
    scratch_types=[
        pltpu.VMEM((UPT, U), jnp.int32),
        pltpu.VMEM((UPT, U), jnp.int32),
        pltpu.VMEM((UPT, U), jnp.float32),
        pltpu.VMEM((U,), jnp.float32),
        pltpu.VMEM((NP,), jnp.float32),
        pltpu.VMEM_SHARED((NP,), jnp.float32),
        pltpu.VMEM_SHARED((NP,), jnp.float32),
        pltpu.SemaphoreType.DMA,
    ],
    compiler_params=_SC_PARAMS,
)
def _deg_kernel(dst_h, gdst_h, w_h, out1_h, out2_h,
                didx, gdidx, wbuf, ones_v, zbuf, acc1, acc2, sem):
    c = lax.axis_index("c")
    s = lax.axis_index("s")
    wid = c * NS + s
    tbase = pl.multiple_of(wid * UPT, 8)

    pltpu.sync_copy(dst_h.at[pl.ds(tbase, UPT), :], didx)
    pltpu.sync_copy(gdst_h.at[pl.ds(tbase, UPT), :], gdidx)
    pltpu.sync_copy(w_h.at[pl.ds(tbase, UPT), :], wbuf)

    for i in range(U // 16):
        ones_v[pl.ds(i * 16, 16)] = jnp.ones((16,), jnp.float32)

    @pl.when(s == 0)
    def _():
        def zb(i, carry):
            zbuf[pl.ds(i * 16, 16)] = jnp.zeros((16,), jnp.float32)
            return carry
        lax.fori_loop(0, NP // 16, zb, 0)
        pltpu.sync_copy(zbuf, acc1)
        pltpu.sync_copy(zbuf, acc2)

    plsc.subcore_barrier()

    def fire(rd):
        for j in range(GD):
            u = rd * GD + j
            pltpu.async_copy(ones_v, acc1.at[didx.at[u]], sem, add=True)
            pltpu.async_copy(wbuf.at[u], acc2.at[gdidx.at[u]], sem, add=True)

    def drain(rd):
        for j in range(GD):
            u = rd * GD + j
            pltpu.make_async_copy(ones_v, acc1.at[didx.at[u]], sem).wait()
            pltpu.make_async_copy(wbuf.at[u], acc2.at[gdidx.at[u]], sem).wait()

    def round_body(rd, carry):
        fire(rd)
        @pl.when(rd > 0)
        def _():
            drain(rd - 1)
        return carry
    lax.fori_loop(0, ROUNDS_D, round_body, 0)
    drain(ROUNDS_D - 1)

    plsc.subcore_barrier()

    @pl.when(s == 0)
    def _():
        pltpu.sync_copy(acc1.at[pl.ds(0, N)], out1_h.at[c])
        pltpu.sync_copy(acc2.at[pl.ds(0, N)], out2_h.at[c])


def _make_scatter_pass(F, weighted, G=1, spmem_table=False):
    NU = UPT
    scratch = [
        pltpu.VMEM((UPT, U), jnp.int32),
        pltpu.VMEM((UPT, U), jnp.int32),
        pltpu.VMEM((U, F), jnp.float32),
        pltpu.VMEM((U, F), jnp.float32),
        pltpu.VMEM((U, F), jnp.float32),
        pltpu.VMEM((ZB, F), jnp.float32),
    ]
    if weighted:
        scratch.append(pltpu.VMEM((UPT, U), jnp.float32))
    scratch.append(pltpu.VMEM_SHARED((NP, F), jnp.float32))
    if spmem_table:
        scratch.append(pltpu.VMEM_SHARED((N, F), jnp.float32))
    for _ in range(6):
        scratch.append(pltpu.SemaphoreType.DMA)

    def body(table_h, src_h, dst_h, *rest):
        if weighted:
            (w_h, out_h, sidx, didx, b0, b1, b2, zbuf, wbuf, acc,
             *tail) = rest
        else:
            (out_h, sidx, didx, b0, b1, b2, zbuf, acc, *tail) = rest
            wbuf = None
        if spmem_table:
            (tbl, g0, g1, g2, s0, s1, s2) = tail
        else:
            (g0, g1, g2, s0, s1, s2) = tail
            tbl = None
        bufs = (b0, b1, b2)
        gsems = (g0, g1, g2)
        ssems = (s0, s1, s2)
        c = lax.axis_index("c")
        s = lax.axis_index("s")
        wid = c * NS + s
        tbase = pl.multiple_of(wid * UPT, 8)
        row_lo = pl.multiple_of(s * RB, 8)

        pltpu.sync_copy(src_h.at[pl.ds(tbase, UPT), :], sidx)
        pltpu.sync_copy(dst_h.at[pl.ds(tbase, UPT), :], didx)
        if weighted:
            pltpu.sync_copy(w_h.at[pl.ds(tbase, UPT), :], wbuf)

        tsrc = tbl if spmem_table else table_h

        def fire_g(u, o):
            pltpu.async_copy(tsrc.at[sidx.at[u]], bufs[o], gsems[o])

        def drain_g(u, o):
            pltpu.make_async_copy(tsrc.at[sidx.at[u]], bufs[o],
                                  gsems[o]).wait()

        def fire_s(u, o):
            pltpu.async_copy(bufs[o], acc.at[didx.at[u]], ssems[o], add=True)

        def drain_s(u, o):
            pltpu.make_async_copy(bufs[o], acc.at[didx.at[u]],
                                  ssems[o]).wait()

        def mul(u, o):
            rows = bufs[o]
            def mul_blk(bk, carry2):
                off = bk * 16
                wv16 = wbuf[u, pl.ds(off, 16)]
                for r16 in range(16):
                    wv = wv16[r16]
                    for cg in range(F // 16):
                        cur = rows[off + r16, pl.ds(cg * 16, 16)]
                        rows[off + r16, pl.ds(cg * 16, 16)] = cur * wv
                return carry2
            lax.fori_loop(0, U // 16, mul_blk, 0)

        def process(u, o):
            drain_g(u, o)
            if weighted:
                mul(u, o)
            fire_s(u, o)

        if spmem_table:
            pltpu.sync_copy(table_h.at[pl.ds(row_lo, RB), :],
                            tbl.at[pl.ds(row_lo, RB), :])
            @pl.when(s == NS - 1)
            def _():
                pltpu.sync_copy(table_h.at[pl.ds(NS * RB, N - NS * RB), :],
                                tbl.at[pl.ds(NS * RB, N - NS * RB), :])
        else:
            fire_g(0, 0)

        def zb(r, carry):
            for cg in range(F // 16):
                zbuf[r, pl.ds(cg * 16, 16)] = jnp.zeros((16,), jnp.float32)
            return carry
        lax.fori_loop(0, ZB, zb, 0)
        for t in range(RB // ZB):
            pltpu.sync_copy(zbuf, acc.at[pl.ds(row_lo + t * ZB, ZB), :])
        rem = RB - (RB // ZB) * ZB
        if rem:
            pltpu.sync_copy(zbuf.at[pl.ds(0, rem), :],
                            acc.at[pl.ds(row_lo + RB - rem, rem), :])
        ztail = NP - NS * RB
        @pl.when(s == NS - 1)
        def _():
            pltpu.sync_copy(zbuf.at[pl.ds(0, ztail), :],
                            acc.at[pl.ds(NS * RB, ztail), :])
        plsc.subcore_barrier()

        if spmem_table:
            fire_g(0, 0)
        fire_g(1, 1)
        process(0, 0)
        fire_g(2, 2)
        process(1, 1)
        def round_body(i, carry):
            k = 3 * i
            drain_s(k - 3, 0)
            fire_g(k, 0)
            process(k - 1, 2)
            drain_s(k - 2, 1)
            fire_g(k + 1, 1)
            process(k, 0)
            drain_s(k - 1, 2)
            fire_g(k + 2, 2)
            process(k + 1, 1)
            return carry
        steady_end = 3 * ((NU - 1) // 3)
        lax.fori_loop(1, (NU - 1) // 3, round_body, 0)
        for k in range(steady_end, NU):
            drain_s(k - 3, (k - 3) % 3)
            fire_g(k, k % 3)
            process(k - 1, (k - 1) % 3)
        drain_s(NU - 3, (NU - 3) % 3)
        process(NU - 1, (NU - 1) % 3)
        drain_s(NU - 2, (NU - 2) % 3)
        drain_s(NU - 1, (NU - 1) % 3)

        plsc.subcore_barrier()
        pltpu.sync_copy(acc.at[pl.ds(row_lo, RB), :],
                        out_h.at[c, pl.ds(row_lo, RB), :])
        dtail = N - NS * RB
        @pl.when(s == NS - 1)
        def _():
            pltpu.sync_copy(acc.at[pl.ds(NS * RB, dtail), :],
                            out_h.at[c, pl.ds(NS * RB, dtail), :])

    return pl.kernel(body,
                     out_type=jax.ShapeDtypeStruct((NC, N, F), jnp.float32),
                     mesh=_mesh(), scratch_types=scratch,
                     compiler_params=_SC_PARAMS)


_diff_pass = _make_scatter_pass(HID, weighted=False)
_gcn1_pass = _make_scatter_pass(HID, weighted=True)
_gcn2_pass = _make_scatter_pass(C, weighted=True)


def _safe_rsqrt(deg):
    return jnp.where(deg > 0, lax.rsqrt(jnp.where(deg > 0, deg, 1.0)), 0.0)


def _prep_body(d1_ref, d2_ref, x_ref, w1_ref, dis_ref, dis2_ref, z0_ref, p0_ref):
    deg = d1_ref[0] + d1_ref[1]
    dis = _safe_rsqrt(deg)
    deg2 = d2_ref[0] + d2_ref[1] + 1.0
    dis2 = _safe_rsqrt(deg2)
    dis_ref[...] = dis
    dis2_ref[...] = dis2
    z0 = jnp.dot(x_ref[...], w1_ref[...], preferred_element_type=jnp.float32)
    z0_ref[...] = z0
    p0_ref[...] = dis * z0


def _step_body(sp_ref, dis_ref, acc_ref, coef_ref, accout_ref, p_ref):
    h = dis_ref[...] * (sp_ref[0] + sp_ref[1])
    accout_ref[...] = acc_ref[...] + coef_ref[0, 0] * h
    p_ref[...] = dis_ref[...] * h


def _qprep_body(acc_ref, dis2_ref, q_ref):
    q_ref[...] = dis2_ref[...] * (acc_ref[...] * E_NEG_T)


def _gcn1_body(tp_ref, q_ref, dis2_ref, b1_ref, w2_ref, q2_ref):
    l1 = dis2_ref[...] * (tp_ref[0] + tp_ref[1] + q_ref[...]) + b1_ref[...]
    r = jnp.maximum(l1, 0.0)
    z2 = jnp.dot(r, w2_ref[...], preferred_element_type=jnp.float32)
    q2_ref[...] = dis2_ref[...] * z2


def _gcn2_body(up_ref, q2_ref, dis2_ref, b2_ref, out_ref):
    l2 = dis2_ref[...] * (up_ref[0] + up_ref[1] + q2_ref[...]) + b2_ref[...]
    m = jnp.max(l2, axis=1, keepdims=True)
    e = l2 - m
    lse = jnp.log(jnp.sum(jnp.exp(e), axis=1, keepdims=True))
    out_ref[...] = e - lse


def _tc(body, out_shape, *args):
    return pl.pallas_call(body, out_shape=out_shape)(*args)


def kernel(x, edge_index, gdc_edge_index, gdc_edge_attr, W1, b1, W2, b2):
    pad_src = jnp.asarray(_PAD_SRC)
    pad_dst = jnp.asarray(_PAD_DST)
    pad_w = jnp.asarray(_PAD_W)
    src = jnp.concatenate([edge_index[0], pad_src]).reshape(ROWS, U)
    dst = jnp.concatenate([edge_index[1], pad_dst]).reshape(ROWS, U)
    gsrc = jnp.concatenate([gdc_edge_index[0], pad_src]).reshape(ROWS, U)
    gdst = jnp.concatenate([gdc_edge_index[1], pad_dst]).reshape(ROWS, U)
    wflat = jnp.concatenate([gdc_edge_attr, pad_w]).reshape(ROWS, U)

    degp, deg2p = _deg_kernel(dst, gdst, wflat)
    degp = degp.reshape(NC, N, 1)
    deg2p = deg2p.reshape(NC, N, 1)

    f32 = jnp.float32
    dis, dis2, z0, p = _tc(
        _prep_body,
        (jax.ShapeDtypeStruct((N, 1), f32), jax.ShapeDtypeStruct((N, 1), f32),
         jax.ShapeDtypeStruct((N, HID), f32), jax.ShapeDtypeStruct((N, HID), f32)),
        degp, deg2p, x, W1)

    acc = z0
    coef = 1.0
    for k in range(1, K_TAYLOR + 1):
        coef = coef * T / k
        sp = _diff_pass(p, src, dst)
        coef_k = jnp.full((1, 1), coef, f32)
        acc, p = _tc(
            _step_body,
            (jax.ShapeDtypeStruct((N, HID), f32),
             jax.ShapeDtypeStruct((N, HID), f32)),
            sp, dis, acc, coef_k)
    q = _tc(_qprep_body, jax.ShapeDtypeStruct((N, HID), f32),
            acc, dis2)

    tp = _gcn1_pass(q, gsrc, gdst, wflat)
    q2 = _tc(_gcn1_body, jax.ShapeDtypeStruct((N, C), f32),
             tp, q, dis2, b1.reshape(1, HID), W2)

    up = _gcn2_pass(q2, gsrc, gdst, wflat)
    out = _tc(_gcn2_body, jax.ShapeDtypeStruct((N, C), f32),
              up, q2, dis2, b2.reshape(1, C))
    return out


# speedup vs baseline: 1.1623x; 1.1623x over previous
import functools

import jax
import jax.numpy as jnp
import numpy as np
from jax import lax
from jax.experimental import pallas as pl
from jax.experimental.pallas import tpu as pltpu
from jax.experimental.pallas import tpu_sc as plsc

N = 10000
E = 320000
D_IN = 128
HID = 64
C = 32
T = 3.0
K_TAYLOR = 5
E_NEG_T = float(np.exp(-T))

NC = 2
NS = 16
NW = NC * NS
U = 128
E2 = 327680
PAD = E2 - E
ROWS = E2 // U
UPT = ROWS // NW
GD = 8
ROUNDS_D = UPT // GD
NP = N + 16
RB = 624
ZB = 64

_PAD_SRC = np.arange(PAD, dtype=np.int32) % 16
_PAD_DST = N + (np.arange(PAD, dtype=np.int32) % 16)
_PAD_W = np.zeros((PAD,), dtype=np.float32)


def _mesh():
    return plsc.VectorSubcoreMesh(core_axis_name="c", subcore_axis_name="s")


_SC_PARAMS = pltpu.CompilerParams(use_tc_tiling_on_sc=False)


@functools.partial(
    pl.kernel,
    out_type=(
        jax.ShapeDtypeStruct((NC, N), jnp.float32),
        jax.ShapeDtypeStruct((NC, N), jnp.float32),
    ),
    mesh=_mesh(),
    scratch_types=[
        pltpu.VMEM((UPT, U), jnp.int32),
        pltpu.VMEM((UPT, U), jnp.int32),
        pltpu.VMEM((UPT, U), jnp.float32),
        pltpu.VMEM((U,), jnp.float32),
        pltpu.VMEM((NP,), jnp.float32),
        pltpu.VMEM_SHARED((NP,), jnp.float32),
        pltpu.VMEM_SHARED((NP,), jnp.float32),
        pltpu.SemaphoreType.DMA,
    ],
    compiler_params=_SC_PARAMS,
)
def _deg_kernel(dst_h, gdst_h, w_h, out1_h, out2_h,
                didx, gdidx, wbuf, ones_v, zbuf, acc1, acc2, sem):
    c = lax.axis_index("c")
    s = lax.axis_index("s")
    wid = c * NS + s
    tbase = pl.multiple_of(wid * UPT, 8)

    pltpu.sync_copy(dst_h.at[pl.ds(tbase, UPT), :], didx)
    pltpu.sync_copy(gdst_h.at[pl.ds(tbase, UPT), :], gdidx)
    pltpu.sync_copy(w_h.at[pl.ds(tbase, UPT), :], wbuf)

    for i in range(U // 16):
        ones_v[pl.ds(i * 16, 16)] = jnp.ones((16,), jnp.float32)

    @pl.when(s == 0)
    def _():
        def zb(i, carry):
            zbuf[pl.ds(i * 16, 16)] = jnp.zeros((16,), jnp.float32)
            return carry
        lax.fori_loop(0, NP // 16, zb, 0)
        pltpu.sync_copy(zbuf, acc1)
        pltpu.sync_copy(zbuf, acc2)

    plsc.subcore_barrier()

    def fire(rd):
        for j in range(GD):
            u = rd * GD + j
            pltpu.async_copy(ones_v, acc1.at[didx.at[u]], sem, add=True)
            pltpu.async_copy(wbuf.at[u], acc2.at[gdidx.at[u]], sem, add=True)

    def drain(rd):
        for j in range(GD):
            u = rd * GD + j
            pltpu.make_async_copy(ones_v, acc1.at[didx.at[u]], sem).wait()
            pltpu.make_async_copy(wbuf.at[u], acc2.at[gdidx.at[u]], sem).wait()

    def round_body(rd, carry):
        fire(rd)
        @pl.when(rd > 0)
        def _():
            drain(rd - 1)
        return carry
    lax.fori_loop(0, ROUNDS_D, round_body, 0)
    drain(ROUNDS_D - 1)

    plsc.subcore_barrier()

    @pl.when(s == 0)
    def _():
        pltpu.sync_copy(acc1.at[pl.ds(0, N)], out1_h.at[c])
        pltpu.sync_copy(acc2.at[pl.ds(0, N)], out2_h.at[c])


def _make_scatter_pass(F, weighted, G=1, spmem_table=False):
    NU = UPT
    scratch = [
        pltpu.VMEM((UPT, U), jnp.int32),
        pltpu.VMEM((UPT, U), jnp.int32),
        pltpu.VMEM((U, F), jnp.float32),
        pltpu.VMEM((U, F), jnp.float32),
        pltpu.VMEM((U, F), jnp.float32),
        pltpu.VMEM((ZB, F), jnp.float32),
    ]
    if weighted:
        scratch.append(pltpu.VMEM((UPT, U), jnp.float32))
    scratch.append(pltpu.VMEM_SHARED((NP, F), jnp.float32))
    if spmem_table:
        scratch.append(pltpu.VMEM_SHARED((N, F), jnp.float32))
    for _ in range(6):
        scratch.append(pltpu.SemaphoreType.DMA)

    def body(table_h, src_h, dst_h, *rest):
        if weighted:
            (w_h, out_h, sidx, didx, b0, b1, b2, zbuf, wbuf, acc,
             *tail) = rest
        else:
            (out_h, sidx, didx, b0, b1, b2, zbuf, acc, *tail) = rest
            wbuf = None
        if spmem_table:
            (tbl, g0, g1, g2, s0, s1, s2) = tail
        else:
            (g0, g1, g2, s0, s1, s2) = tail
            tbl = None
        bufs = (b0, b1, b2)
        gsems = (g0, g1, g2)
        ssems = (s0, s1, s2)
        c = lax.axis_index("c")
        s = lax.axis_index("s")
        wid = c * NS + s
        tbase = pl.multiple_of(wid * UPT, 8)
        row_lo = pl.multiple_of(s * RB, 8)

        pltpu.sync_copy(src_h.at[pl.ds(tbase, UPT), :], sidx)
        pltpu.sync_copy(dst_h.at[pl.ds(tbase, UPT), :], didx)
        if weighted:
            pltpu.sync_copy(w_h.at[pl.ds(tbase, UPT), :], wbuf)

        tsrc = tbl if spmem_table else table_h

        def fire_g(u, o):
            pltpu.async_copy(tsrc.at[sidx.at[u]], bufs[o], gsems[o])

        def drain_g(u, o):
            pltpu.make_async_copy(tsrc.at[sidx.at[u]], bufs[o],
                                  gsems[o]).wait()

        def fire_s(u, o):
            pltpu.async_copy(bufs[o], acc.at[didx.at[u]], ssems[o], add=True)

        def drain_s(u, o):
            pltpu.make_async_copy(bufs[o], acc.at[didx.at[u]],
                                  ssems[o]).wait()

        def mul(u, o):
            rows = bufs[o]
            @plsc.parallel_loop(0, U // 16, unroll=2)
            def mul_blk(bk):
                off = bk * 16
                wv16 = wbuf[u, pl.ds(off, 16)]
                for r16 in range(16):
                    wv = wv16[r16]
                    for cg in range(F // 16):
                        cur = rows[off + r16, pl.ds(cg * 16, 16)]
                        rows[off + r16, pl.ds(cg * 16, 16)] = cur * wv

        def process(u, o):
            drain_g(u, o)
            if weighted:
                mul(u, o)
            fire_s(u, o)

        if spmem_table:
            pltpu.sync_copy(table_h.at[pl.ds(row_lo, RB), :],
                            tbl.at[pl.ds(row_lo, RB), :])
            @pl.when(s == NS - 1)
            def _():
                pltpu.sync_copy(table_h.at[pl.ds(NS * RB, N - NS * RB), :],
                                tbl.at[pl.ds(NS * RB, N - NS * RB), :])
        else:
            fire_g(0, 0)

        def zb(r, carry):
            for cg in range(F // 16):
                zbuf[r, pl.ds(cg * 16, 16)] = jnp.zeros((16,), jnp.float32)
            return carry
        lax.fori_loop(0, ZB, zb, 0)
        for t in range(RB // ZB):
            pltpu.sync_copy(zbuf, acc.at[pl.ds(row_lo + t * ZB, ZB), :])
        rem = RB - (RB // ZB) * ZB
        if rem:
            pltpu.sync_copy(zbuf.at[pl.ds(0, rem), :],
                            acc.at[pl.ds(row_lo + RB - rem, rem), :])
        ztail = NP - NS * RB
        @pl.when(s == NS - 1)
        def _():
            pltpu.sync_copy(zbuf.at[pl.ds(0, ztail), :],
                            acc.at[pl.ds(NS * RB, ztail), :])
        plsc.subcore_barrier()

        if spmem_table:
            fire_g(0, 0)
        fire_g(1, 1)
        process(0, 0)
        fire_g(2, 2)
        process(1, 1)
        def round_body(i, carry):
            k = 3 * i
            drain_s(k - 3, 0)
            fire_g(k, 0)
            process(k - 1, 2)
            drain_s(k - 2, 1)
            fire_g(k + 1, 1)
            process(k, 0)
            drain_s(k - 1, 2)
            fire_g(k + 2, 2)
            process(k + 1, 1)
            return carry
        steady_end = 3 * ((NU - 1) // 3)
        lax.fori_loop(1, (NU - 1) // 3, round_body, 0)
        for k in range(steady_end, NU):
            drain_s(k - 3, (k - 3) % 3)
            fire_g(k, k % 3)
            process(k - 1, (k - 1) % 3)
        drain_s(NU - 3, (NU - 3) % 3)
        process(NU - 1, (NU - 1) % 3)
        drain_s(NU - 2, (NU - 2) % 3)
        drain_s(NU - 1, (NU - 1) % 3)

        plsc.subcore_barrier()
        pltpu.sync_copy(acc.at[pl.ds(row_lo, RB), :],
                        out_h.at[c, pl.ds(row_lo, RB), :])
        dtail = N - NS * RB
        @pl.when(s == NS - 1)
        def _():
            pltpu.sync_copy(acc.at[pl.ds(NS * RB, dtail), :],
                            out_h.at[c, pl.ds(NS * RB, dtail), :])

    return pl.kernel(body,
                     out_type=jax.ShapeDtypeStruct((NC, N, F), jnp.float32),
                     mesh=_mesh(), scratch_types=scratch,
                     compiler_params=_SC_PARAMS)


_diff_pass = _make_scatter_pass(HID, weighted=False, spmem_table=True)
_gcn1_pass = _make_scatter_pass(HID, weighted=True)
_gcn2_pass = _make_scatter_pass(C, weighted=True)


def _safe_rsqrt(deg):
    return jnp.where(deg > 0, lax.rsqrt(jnp.where(deg > 0, deg, 1.0)), 0.0)


def _prep_body(d1_ref, d2_ref, x_ref, w1_ref, dis_ref, dis2_ref, z0_ref, p0_ref):
    deg = d1_ref[0] + d1_ref[1]
    dis = _safe_rsqrt(deg)
    deg2 = d2_ref[0] + d2_ref[1] + 1.0
    dis2 = _safe_rsqrt(deg2)
    dis_ref[...] = dis
    dis2_ref[...] = dis2
    z0 = jnp.dot(x_ref[...], w1_ref[...], preferred_element_type=jnp.float32)
    z0_ref[...] = z0
    p0_ref[...] = dis * z0


def _step_body(sp_ref, dis_ref, acc_ref, coef_ref, accout_ref, p_ref):
    h = dis_ref[...] * (sp_ref[0] + sp_ref[1])
    accout_ref[...] = acc_ref[...] + coef_ref[0, 0] * h
    p_ref[...] = dis_ref[...] * h


def _qprep_body(acc_ref, dis2_ref, q_ref):
    q_ref[...] = dis2_ref[...] * (acc_ref[...] * E_NEG_T)


def _gcn1_body(tp_ref, q_ref, dis2_ref, b1_ref, w2_ref, q2_ref):
    l1 = dis2_ref[...] * (tp_ref[0] + tp_ref[1] + q_ref[...]) + b1_ref[...]
    r = jnp.maximum(l1, 0.0)
    z2 = jnp.dot(r, w2_ref[...], preferred_element_type=jnp.float32)
    q2_ref[...] = dis2_ref[...] * z2


def _gcn2_body(up_ref, q2_ref, dis2_ref, b2_ref, out_ref):
    l2 = dis2_ref[...] * (up_ref[0] + up_ref[1] + q2_ref[...]) + b2_ref[...]
    m = jnp.max(l2, axis=1, keepdims=True)
    e = l2 - m
    lse = jnp.log(jnp.sum(jnp.exp(e), axis=1, keepdims=True))
    out_ref[...] = e - lse


def _tc(body, out_shape, *args):
    return pl.pallas_call(body, out_shape=out_shape)(*args)


def kernel(x, edge_index, gdc_edge_index, gdc_edge_attr, W1, b1, W2, b2):
    pad_src = jnp.asarray(_PAD_SRC)
    pad_dst = jnp.asarray(_PAD_DST)
    pad_w = jnp.asarray(_PAD_W)
    src = jnp.concatenate([edge_index[0], pad_src]).reshape(ROWS, U)
    dst = jnp.concatenate([edge_index[1], pad_dst]).reshape(ROWS, U)
    gsrc = jnp.concatenate([gdc_edge_index[0], pad_src]).reshape(ROWS, U)
    gdst = jnp.concatenate([gdc_edge_index[1], pad_dst]).reshape(ROWS, U)
    wflat = jnp.concatenate([gdc_edge_attr, pad_w]).reshape(ROWS, U)

    degp, deg2p = _deg_kernel(dst, gdst, wflat)
    degp = degp.reshape(NC, N, 1)
    deg2p = deg2p.reshape(NC, N, 1)

    f32 = jnp.float32
    dis, dis2, z0, p = _tc(
        _prep_body,
        (jax.ShapeDtypeStruct((N, 1), f32), jax.ShapeDtypeStruct((N, 1), f32),
         jax.ShapeDtypeStruct((N, HID), f32), jax.ShapeDtypeStruct((N, HID), f32)),
        degp, deg2p, x, W1)

    acc = z0
    coef = 1.0
    for k in range(1, K_TAYLOR + 1):
        coef = coef * T / k
        sp = _diff_pass(p, src, dst)
        coef_k = jnp.full((1, 1), coef, f32)
        acc, p = _tc(
            _step_body,
            (jax.ShapeDtypeStruct((N, HID), f32),
             jax.ShapeDtypeStruct((N, HID), f32)),
            sp, dis, acc, coef_k)
    q = _tc(_qprep_body, jax.ShapeDtypeStruct((N, HID), f32),
            acc, dis2)

    tp = _gcn1_pass(q, gsrc, gdst, wflat)
    q2 = _tc(_gcn1_body, jax.ShapeDtypeStruct((N, C), f32),
             tp, q, dis2, b1.reshape(1, HID), W2)

    up = _gcn2_pass(q2, gsrc, gdst, wflat)
    out = _tc(_gcn2_body, jax.ShapeDtypeStruct((N, C), f32),
              up, q2, dis2, b2.reshape(1, C))
    return out


# speedup vs baseline: 1.1895x; 1.0233x over previous
import functools

import jax
import jax.numpy as jnp
import numpy as np
from jax import lax
from jax.experimental import pallas as pl
from jax.experimental.pallas import tpu as pltpu
from jax.experimental.pallas import tpu_sc as plsc

N = 10000
E = 320000
D_IN = 128
HID = 64
C = 32
T = 3.0
K_TAYLOR = 5
E_NEG_T = float(np.exp(-T))

NC = 2
NS = 16
NW = NC * NS
U = 128
E2 = 327680
PAD = E2 - E
ROWS = E2 // U
UPT = ROWS // NW
GD = 8
ROUNDS_D = UPT // GD
NP = N + 16
RB = 624
ZB = 64

_PAD_SRC = np.arange(PAD, dtype=np.int32) % 16
_PAD_DST = N + (np.arange(PAD, dtype=np.int32) % 16)
_PAD_W = np.zeros((PAD,), dtype=np.float32)


def _mesh():
    return plsc.VectorSubcoreMesh(core_axis_name="c", subcore_axis_name="s")


_SC_PARAMS = pltpu.CompilerParams(use_tc_tiling_on_sc=False)


@functools.partial(
    pl.kernel,
    out_type=(
        jax.ShapeDtypeStruct((NC, N), jnp.float32),
        jax.ShapeDtypeStruct((NC, N), jnp.float32),
    ),
    mesh=_mesh(),
    scratch_types=[
        pltpu.VMEM((UPT, U), jnp.int32),
        pltpu.VMEM((UPT, U), jnp.int32),
        pltpu.VMEM((UPT, U), jnp.float32),
        pltpu.VMEM((U,), jnp.float32),
        pltpu.VMEM((NP,), jnp.float32),
        pltpu.VMEM_SHARED((NP,), jnp.float32),
        pltpu.VMEM_SHARED((NP,), jnp.float32),
        pltpu.SemaphoreType.DMA,
    ],
    compiler_params=_SC_PARAMS,
)
def _deg_kernel(dst_h, gdst_h, w_h, out1_h, out2_h,
                didx, gdidx, wbuf, ones_v, zbuf, acc1, acc2, sem):
    c = lax.axis_index("c")
    s = lax.axis_index("s")
    wid = c * NS + s
    tbase = pl.multiple_of(wid * UPT, 8)

    idx_copies = [
        pltpu.async_copy(dst_h.at[pl.ds(tbase, UPT), :], didx, sem),
        pltpu.async_copy(gdst_h.at[pl.ds(tbase, UPT), :], gdidx, sem),
        pltpu.async_copy(w_h.at[pl.ds(tbase, UPT), :], wbuf, sem),
    ]

    for i in range(U // 16):
        ones_v[pl.ds(i * 16, 16)] = jnp.ones((16,), jnp.float32)

    @pl.when(s == 0)
    def _():
        def zb(i, carry):
            zbuf[pl.ds(i * 16, 16)] = jnp.zeros((16,), jnp.float32)
            return carry
        lax.fori_loop(0, NP // 16, zb, 0)
        pltpu.sync_copy(zbuf, acc1)
        pltpu.sync_copy(zbuf, acc2)

    for d in idx_copies:
        d.wait()
    plsc.subcore_barrier()

    def fire(rd):
        for j in range(GD):
            u = rd * GD + j
            pltpu.async_copy(ones_v, acc1.at[didx.at[u]], sem, add=True)
            pltpu.async_copy(wbuf.at[u], acc2.at[gdidx.at[u]], sem, add=True)

    def drain(rd):
        for j in range(GD):
            u = rd * GD + j
            pltpu.make_async_copy(ones_v, acc1.at[didx.at[u]], sem).wait()
            pltpu.make_async_copy(wbuf.at[u], acc2.at[gdidx.at[u]], sem).wait()

    def round_body(rd, carry):
        fire(rd)
        @pl.when(rd > 0)
        def _():
            drain(rd - 1)
        return carry
    lax.fori_loop(0, ROUNDS_D, round_body, 0)
    drain(ROUNDS_D - 1)

    plsc.subcore_barrier()

    @pl.when(s == 0)
    def _():
        pltpu.sync_copy(acc1.at[pl.ds(0, N)], out1_h.at[c])
        pltpu.sync_copy(acc2.at[pl.ds(0, N)], out2_h.at[c])


def _make_scatter_pass(F, weighted, G=1, spmem_table=False):
    NU = UPT
    scratch = [
        pltpu.VMEM((UPT, U), jnp.int32),
        pltpu.VMEM((UPT, U), jnp.int32),
        pltpu.VMEM((U, F), jnp.float32),
        pltpu.VMEM((U, F), jnp.float32),
        pltpu.VMEM((U, F), jnp.float32),
        pltpu.VMEM((ZB, F), jnp.float32),
    ]
    if weighted:
        scratch.append(pltpu.VMEM((UPT, U), jnp.float32))
    scratch.append(pltpu.VMEM_SHARED((NP, F), jnp.float32))
    if spmem_table:
        scratch.append(pltpu.VMEM_SHARED((N, F), jnp.float32))
    for _ in range(6):
        scratch.append(pltpu.SemaphoreType.DMA)

    def body(table_h, src_h, dst_h, *rest):
        if weighted:
            (w_h, out_h, sidx, didx, b0, b1, b2, zbuf, wbuf, acc,
             *tail) = rest
        else:
            (out_h, sidx, didx, b0, b1, b2, zbuf, acc, *tail) = rest
            wbuf = None
        if spmem_table:
            (tbl, g0, g1, g2, s0, s1, s2) = tail
        else:
            (g0, g1, g2, s0, s1, s2) = tail
            tbl = None
        bufs = (b0, b1, b2)
        gsems = (g0, g1, g2)
        ssems = (s0, s1, s2)
        c = lax.axis_index("c")
        s = lax.axis_index("s")
        wid = c * NS + s
        tbase = pl.multiple_of(wid * UPT, 8)
        row_lo = pl.multiple_of(s * RB, 8)

        idx_copies = [
            pltpu.async_copy(src_h.at[pl.ds(tbase, UPT), :], sidx, s0),
            pltpu.async_copy(dst_h.at[pl.ds(tbase, UPT), :], didx, s0),
        ]
        if weighted:
            idx_copies.append(
                pltpu.async_copy(w_h.at[pl.ds(tbase, UPT), :], wbuf, s0))

        tsrc = tbl if spmem_table else table_h

        def fire_g(u, o):
            pltpu.async_copy(tsrc.at[sidx.at[u]], bufs[o], gsems[o])

        def drain_g(u, o):
            pltpu.make_async_copy(tsrc.at[sidx.at[u]], bufs[o],
                                  gsems[o]).wait()

        def fire_s(u, o):
            pltpu.async_copy(bufs[o], acc.at[didx.at[u]], ssems[o], add=True)

        def drain_s(u, o):
            pltpu.make_async_copy(bufs[o], acc.at[didx.at[u]],
                                  ssems[o]).wait()

        def mul(u, o):
            rows = bufs[o]
            @plsc.parallel_loop(0, U // 16, unroll=2)
            def mul_blk(bk):
                off = bk * 16
                wv16 = wbuf[u, pl.ds(off, 16)]
                for r16 in range(16):
                    wv = wv16[r16]
                    for cg in range(F // 16):
                        cur = rows[off + r16, pl.ds(cg * 16, 16)]
                        rows[off + r16, pl.ds(cg * 16, 16)] = cur * wv

        def process(u, o):
            drain_g(u, o)
            if weighted:
                mul(u, o)
            fire_s(u, o)

        if spmem_table:
            pltpu.sync_copy(table_h.at[pl.ds(row_lo, RB), :],
                            tbl.at[pl.ds(row_lo, RB), :])
            @pl.when(s == NS - 1)
            def _():
                pltpu.sync_copy(table_h.at[pl.ds(NS * RB, N - NS * RB), :],
                                tbl.at[pl.ds(NS * RB, N - NS * RB), :])

        def zb(r, carry):
            for cg in range(F // 16):
                zbuf[r, pl.ds(cg * 16, 16)] = jnp.zeros((16,), jnp.float32)
            return carry
        lax.fori_loop(0, ZB, zb, 0)
        for t in range(RB // ZB):
            pltpu.sync_copy(zbuf, acc.at[pl.ds(row_lo + t * ZB, ZB), :])
        rem = RB - (RB // ZB) * ZB
        if rem:
            pltpu.sync_copy(zbuf.at[pl.ds(0, rem), :],
                            acc.at[pl.ds(row_lo + RB - rem, rem), :])
        ztail = NP - NS * RB
        @pl.when(s == NS - 1)
        def _():
            pltpu.sync_copy(zbuf.at[pl.ds(0, ztail), :],
                            acc.at[pl.ds(NS * RB, ztail), :])
        for d in idx_copies:
            d.wait()
        plsc.subcore_barrier()

        fire_g(0, 0)
        fire_g(1, 1)
        process(0, 0)
        fire_g(2, 2)
        process(1, 1)
        def round_body(i, carry):
            k = 3 * i
            drain_s(k - 3, 0)
            fire_g(k, 0)
            process(k - 1, 2)
            drain_s(k - 2, 1)
            fire_g(k + 1, 1)
            process(k, 0)
            drain_s(k - 1, 2)
            fire_g(k + 2, 2)
            process(k + 1, 1)
            return carry
        steady_end = 3 * ((NU - 1) // 3)
        lax.fori_loop(1, (NU - 1) // 3, round_body, 0)
        for k in range(steady_end, NU):
            drain_s(k - 3, (k - 3) % 3)
            fire_g(k, k % 3)
            process(k - 1, (k - 1) % 3)
        drain_s(NU - 3, (NU - 3) % 3)
        process(NU - 1, (NU - 1) % 3)
        drain_s(NU - 2, (NU - 2) % 3)
        drain_s(NU - 1, (NU - 1) % 3)

        plsc.subcore_barrier()
        pltpu.sync_copy(acc.at[pl.ds(row_lo, RB), :],
                        out_h.at[c, pl.ds(row_lo, RB), :])
        dtail = N - NS * RB
        @pl.when(s == NS - 1)
        def _():
            pltpu.sync_copy(acc.at[pl.ds(NS * RB, dtail), :],
                            out_h.at[c, pl.ds(NS * RB, dtail), :])

    return pl.kernel(body,
                     out_type=jax.ShapeDtypeStruct((NC, N, F), jnp.float32),
                     mesh=_mesh(), scratch_types=scratch,
                     compiler_params=_SC_PARAMS)


_diff_pass = _make_scatter_pass(HID, weighted=False, spmem_table=True)
_gcn1_pass = _make_scatter_pass(HID, weighted=True)
_gcn2_pass = _make_scatter_pass(C, weighted=True)


def _safe_rsqrt(deg):
    return jnp.where(deg > 0, lax.rsqrt(jnp.where(deg > 0, deg, 1.0)), 0.0)


def _prep_body(d1_ref, d2_ref, x_ref, w1_ref, dis_ref, dis2_ref, z0_ref, p0_ref):
    deg = d1_ref[0] + d1_ref[1]
    dis = _safe_rsqrt(deg)
    deg2 = d2_ref[0] + d2_ref[1] + 1.0
    dis2 = _safe_rsqrt(deg2)
    dis_ref[...] = dis
    dis2_ref[...] = dis2
    z0 = jnp.dot(x_ref[...], w1_ref[...], preferred_element_type=jnp.float32)
    z0_ref[...] = z0
    p0_ref[...] = dis * z0


def _step_body(sp_ref, dis_ref, acc_ref, coef_ref, accout_ref, p_ref):
    h = dis_ref[...] * (sp_ref[0] + sp_ref[1])
    accout_ref[...] = acc_ref[...] + coef_ref[0, 0] * h
    p_ref[...] = dis_ref[...] * h


def _qprep_body(acc_ref, dis2_ref, q_ref):
    q_ref[...] = dis2_ref[...] * (acc_ref[...] * E_NEG_T)


def _gcn1_body(tp_ref, q_ref, dis2_ref, b1_ref, w2_ref, q2_ref):
    l1 = dis2_ref[...] * (tp_ref[0] + tp_ref[1] + q_ref[...]) + b1_ref[...]
    r = jnp.maximum(l1, 0.0)
    z2 = jnp.dot(r, w2_ref[...], preferred_element_type=jnp.float32)
    q2_ref[...] = dis2_ref[...] * z2


def _gcn2_body(up_ref, q2_ref, dis2_ref, b2_ref, out_ref):
    l2 = dis2_ref[...] * (up_ref[0] + up_ref[1] + q2_ref[...]) + b2_ref[...]
    m = jnp.max(l2, axis=1, keepdims=True)
    e = l2 - m
    lse = jnp.log(jnp.sum(jnp.exp(e), axis=1, keepdims=True))
    out_ref[...] = e - lse


def _tc(body, out_shape, *args):
    return pl.pallas_call(body, out_shape=out_shape)(*args)


def kernel(x, edge_index, gdc_edge_index, gdc_edge_attr, W1, b1, W2, b2):
    pad_src = jnp.asarray(_PAD_SRC)
    pad_dst = jnp.asarray(_PAD_DST)
    pad_w = jnp.asarray(_PAD_W)
    src = jnp.concatenate([edge_index[0], pad_src]).reshape(ROWS, U)
    dst = jnp.concatenate([edge_index[1], pad_dst]).reshape(ROWS, U)
    gsrc = jnp.concatenate([gdc_edge_index[0], pad_src]).reshape(ROWS, U)
    gdst = jnp.concatenate([gdc_edge_index[1], pad_dst]).reshape(ROWS, U)
    wflat = jnp.concatenate([gdc_edge_attr, pad_w]).reshape(ROWS, U)

    degp, deg2p = _deg_kernel(dst, gdst, wflat)
    degp = degp.reshape(NC, N, 1)
    deg2p = deg2p.reshape(NC, N, 1)

    f32 = jnp.float32
    dis, dis2, z0, p = _tc(
        _prep_body,
        (jax.ShapeDtypeStruct((N, 1), f32), jax.ShapeDtypeStruct((N, 1), f32),
         jax.ShapeDtypeStruct((N, HID), f32), jax.ShapeDtypeStruct((N, HID), f32)),
        degp, deg2p, x, W1)

    acc = z0
    coef = 1.0
    for k in range(1, K_TAYLOR + 1):
        coef = coef * T / k
        sp = _diff_pass(p, src, dst)
        coef_k = jnp.full((1, 1), coef, f32)
        acc, p = _tc(
            _step_body,
            (jax.ShapeDtypeStruct((N, HID), f32),
             jax.ShapeDtypeStruct((N, HID), f32)),
            sp, dis, acc, coef_k)
    q = _tc(_qprep_body, jax.ShapeDtypeStruct((N, HID), f32),
            acc, dis2)

    tp = _gcn1_pass(q, gsrc, gdst, wflat)
    q2 = _tc(_gcn1_body, jax.ShapeDtypeStruct((N, C), f32),
             tp, q, dis2, b1.reshape(1, HID), W2)

    up = _gcn2_pass(q2, gsrc, gdst, wflat)
    out = _tc(_gcn2_body, jax.ShapeDtypeStruct((N, C), f32),
              up, q2, dis2, b2.reshape(1, C))
    return out
